# Initial kernel scaffold; baseline (speedup 1.0000x reference)
import jax
import jax.numpy as jnp
from jax.experimental import pallas as pl


def kernel(Q, K, V):
    raise NotImplementedError("write your pallas kernel here")


# speedup vs baseline: 6.9590x; 6.9590x over previous
import numpy as np
import jax
import jax.numpy as jnp
from jax.experimental import pallas as pl

_L = 2048
_TOPK = 7
_BN = 256
_BM = 256


def _build_dft_consts():
    t = np.arange(_L, dtype=np.float64)
    om = np.arange(1, _L // 2 + 1, dtype=np.float64)
    th = 2.0 * np.pi * np.outer(om, t) / _L
    gt = np.concatenate([np.cos(th), -np.sin(th)], axis=0)
    c = np.full(_L // 2, 2.0)
    c[-1] = 1.0
    thi = th.T
    hm = np.concatenate([c * np.cos(thi), -c * np.sin(thi)], axis=1) / _L
    return gt.astype(np.float32), hm.astype(np.float32)


def _split_hi_lo(a):
    hi32 = np.asarray(jnp.asarray(a).astype(jnp.bfloat16).astype(jnp.float32))
    lo32 = a - hi32
    return hi32, lo32


_GT_NP, _HM_NP = _build_dft_consts()
_GT_HI32, _GT_LO32 = _split_hi_lo(_GT_NP)
_HM_HI32, _HM_LO32 = _split_hi_lo(_HM_NP)


def _dot3(a_hi, a_lo, b_hi, b_lo):
    acc = jnp.dot(a_hi, b_hi, preferred_element_type=jnp.float32)
    acc += jnp.dot(a_hi, b_lo, preferred_element_type=jnp.float32)
    acc += jnp.dot(a_lo, b_hi, preferred_element_type=jnp.float32)
    return acc


def _split_f32(x):
    hi = x.astype(jnp.bfloat16)
    lo = (x - hi.astype(jnp.float32)).astype(jnp.bfloat16)
    return hi, lo


def _spectrum_kernel(gth_ref, gtl_ref, q_ref, k_ref, xq_ref, xk_ref,
                     dc_ref):
    gth = gth_ref[...]
    gtl = gtl_ref[...]
    qb = q_ref[...]
    kb = k_ref[...]
    qh, ql = _split_f32(qb)
    kh, kl = _split_f32(kb)
    sq = _dot3(gth, gtl, qh, ql)
    sk = _dot3(gth, gtl, kh, kl)
    h = _L // 2
    qr, qi = sq[:h], sq[h:]
    kr, ki = sk[:h], sk[h:]
    re = qr * kr + qi * ki
    im = qi * kr - qr * ki
    x = jnp.concatenate([re, im], axis=0)
    xh, xl = _split_f32(x)
    xq_ref[...] = xh
    xk_ref[...] = xl
    qs = jnp.sum(qb, axis=0, keepdims=True)
    ks = jnp.sum(kb, axis=0, keepdims=True)
    dc8 = jnp.broadcast_to(qs * ks * (1.0 / _L), (8, qb.shape[1]))
    dc_ref[...] = dc8


def _synth_kernel(hmh_ref, hml_ref, xh_ref, xl_ref, v_ref, dc_ref,
                  w_ref, p_ref):
    corr = _dot3(hmh_ref[...], hml_ref[...], xh_ref[...], xl_ref[...])
    dc = dc_ref[0, :][None, :]
    vb = v_ref[...]
    iot = jax.lax.broadcasted_iota(jnp.int32, corr.shape, 0)
    wrows = []
    prows = []
    neg = jnp.float32(-jnp.inf)
    for _ in range(_TOPK):
        m = jnp.max(corr, axis=0, keepdims=True)
        idx = jnp.min(jnp.where(corr == m, iot, _L), axis=0, keepdims=True)
        sel = iot == idx
        pat = jnp.sum(jnp.where(sel, vb, 0.0), axis=0, keepdims=True)
        wrows.append(m + dc)
        prows.append(pat)
        corr = jnp.where(sel, neg, corr)
    zero = jnp.zeros_like(wrows[0])
    w_ref[...] = jnp.concatenate(wrows + [zero], axis=0)
    p_ref[...] = jnp.concatenate(prows + [zero], axis=0)


def _outer_kernel(p_ref, w_ref, o_ref):
    o_ref[...] = jax.lax.dot_general(
        p_ref[...], w_ref[...],
        dimension_numbers=(((0,), (0,)), ((), ())),
        preferred_element_type=jnp.float32,
        precision=jax.lax.Precision.HIGHEST,
    )


def kernel(Q, K, V):
    q0 = Q[0]
    k0 = K[0]
    v0 = V[0]
    gth = jnp.asarray(_GT_HI32).astype(jnp.bfloat16)
    gtl = jnp.asarray(_GT_LO32).astype(jnp.bfloat16)
    hmh = jnp.asarray(_HM_HI32).astype(jnp.bfloat16)
    hml = jnp.asarray(_HM_LO32).astype(jnp.bfloat16)

    nblk = _L // _BN
    full = pl.BlockSpec((_L, _L), lambda j: (0, 0))
    col = pl.BlockSpec((_L, _BN), lambda j: (0, j))
    row8 = pl.BlockSpec((8, _BN), lambda j: (0, j))

    xh, xl, dc = pl.pallas_call(
        _spectrum_kernel,
        grid=(nblk,),
        in_specs=[full, full, col, col],
        out_specs=[
            pl.BlockSpec((_L, _BN), lambda j: (0, j)),
            pl.BlockSpec((_L, _BN), lambda j: (0, j)),
            row8,
        ],
        out_shape=[
            jax.ShapeDtypeStruct((_L, _L), jnp.bfloat16),
            jax.ShapeDtypeStruct((_L, _L), jnp.bfloat16),
            jax.ShapeDtypeStruct((8, _L), jnp.float32),
        ],
    )(gth, gtl, q0, k0)

    wt, pt = pl.pallas_call(
        _synth_kernel,
        grid=(nblk,),
        in_specs=[full, full, col, col, col, row8],
        out_specs=[row8, row8],
        out_shape=[
            jax.ShapeDtypeStruct((8, _L), jnp.float32),
            jax.ShapeDtypeStruct((8, _L), jnp.float32),
        ],
    )(hmh, hml, xh, xl, v0, dc)

    out = pl.pallas_call(
        _outer_kernel,
        grid=(_L // _BM,),
        in_specs=[
            pl.BlockSpec((8, _BM), lambda i: (0, i)),
            pl.BlockSpec((8, _L), lambda i: (0, 0)),
        ],
        out_specs=pl.BlockSpec((_BM, _L), lambda i: (i, 0)),
        out_shape=jax.ShapeDtypeStruct((_L, _L), jnp.float32),
    )(pt, wt)
    return out[None]
